# baseline (speedup 1.0000x reference)
import functools

import jax
import jax.numpy as jnp
from jax import lax
from jax.experimental import pallas as pl
from jax.experimental.pallas import tpu as pltpu
from jax.experimental.pallas import tpu_sc as plsc

B = 128
V = 32768
P = 96
K = 64
W_PART = 0.3

NC = 2
NS = 16
NW = NC * NS
RPW = B // NW
L = 16

NBINS = 1024
CAP = 2048
Q = 4
QV = V // L // Q
QCAP = 1024
QSTR = QCAP + L
MIN_I32 = -(2**31)
BIG_I32 = 2**30


def _skey(v):
    b = lax.bitcast_convert_type(v, jnp.int32)
    return jnp.where(b >= 0, b, b ^ jnp.int32(0x7FFFFFFF))


_mesh = plsc.VectorSubcoreMesh(core_axis_name="c", subcore_axis_name="s")


@functools.partial(
    pl.kernel,
    out_type=(
        jax.ShapeDtypeStruct((B, K), jnp.float32),
        jax.ShapeDtypeStruct((B, K), jnp.int32),
        jax.ShapeDtypeStruct((B, K), jnp.int32),
    ),
    mesh=_mesh,
    compiler_params=pltpu.CompilerParams(needs_layout_passes=False),
    scratch_types=[
        pltpu.VMEM((2 * V,), jnp.float32),
        pltpu.VMEM((NBINS * L,), jnp.int32),
        pltpu.VMEM((Q * QSTR,), jnp.int32),
        pltpu.VMEM((Q * QSTR,), jnp.float32),
        pltpu.VMEM((Q * QSTR,), jnp.int32),
        pltpu.VMEM((P + L,), jnp.float32),
        pltpu.VMEM((P + L,), jnp.int32),
        pltpu.VMEM((P + L,), jnp.int32),
        pltpu.VMEM((P + L,), jnp.float32),
        pltpu.VMEM((P + L,), jnp.int32),
        pltpu.VMEM((P,), jnp.float32),
        pltpu.VMEM((B + L,), jnp.float32),
        pltpu.VMEM((K + L,), jnp.float32),
        pltpu.VMEM((K + L,), jnp.int32),
        pltpu.VMEM((K + L,), jnp.int32),
        pltpu.VMEM((K,), jnp.float32),
        pltpu.VMEM((K,), jnp.int32),
        pltpu.VMEM((K,), jnp.int32),
        pltpu.SemaphoreType.DMA,
    ],
)
def _beam_step(full_hbm, part_hbm, hyp_hbm,
               ovals_hbm, oids_hbm, olids_hbm,
               row_v, hist_v, ck_v, cv_v, ci_v,
               v96, i96, p96, g96, k2, part_v, hyp_v,
               ov, oi, ol, ovx, oix, olx, dsem):
    wid = lax.axis_index("s") * NC + lax.axis_index("c")
    lane = lax.iota(jnp.int32, L)
    lane0 = lane == 0
    pltpu.sync_copy(hyp_hbm, hyp_v.at[pl.ds(0, B)])

    def _splat(x):
        return jnp.full((L,), x)

    pltpu.async_copy(full_hbm.at[wid * RPW], row_v.at[pl.ds(0, V)], dsem)

    def do_row(r, _):
        row = wid * RPW + r
        base = (r % 2) * V
        pltpu.make_async_copy(
            full_hbm.at[row], row_v.at[pl.ds(base, V)], dsem).wait()

        @pl.when(r + 1 < RPW)
        def _prefetch():
            pltpu.async_copy(
                full_hbm.at[row + 1],
                row_v.at[pl.ds(((r + 1) % 2) * V, V)], dsem)

        pltpu.sync_copy(part_hbm.at[row], part_v)

        ones = jnp.ones((L,), jnp.int32)

        def hist_body(i, mx):
            kk = _skey(row_v[pl.ds(base + i * L, L)])
            bins = lax.shift_right_arithmetic(kk, 22) + 512
            return jnp.maximum(mx, jnp.where(bins >= 0, kk, kk))

        mxv = lax.fori_loop(0, V // L, hist_body,
                            jnp.full((L,), MIN_I32, jnp.int32), unroll=4)
        top_bin = lax.shift_right_arithmetic(jnp.max(mxv), 22) + 512

        def t_cond(c):
            b, acc = c
            return jnp.logical_and(acc < P, b > 0)

        def t_body(c):
            b, acc = c
            b = b - 1
            acc = acc + jnp.sum(hist_v[pl.ds(b * L, L)])
            return (b, acc)

        thr_bin, _ = lax.while_loop(t_cond, t_body,
                                    (top_bin + 1, jnp.int32(0)))

        thr_key = lax.shift_left(thr_bin - 512, 22)

        def coll(i, cs):
            ncs = []
            for q in range(Q):
                cq = cs[q]
                v = row_v[pl.ds(base + (q * QV + i) * L, L)]
                kk = _skey(v)
                m = kk >= thr_key
                nm = plsc.all_reduce_population_count(m)[0]
                plsc.store_compressed(ck_v.at[pl.ds(q * QSTR + cq, L)], kk,
                                      mask=m)
                plsc.store_compressed(cv_v.at[pl.ds(q * QSTR + cq, L)], v,
                                      mask=m)
                plsc.store_compressed(ci_v.at[pl.ds(q * QSTR + cq, L)],
                                      (q * QV + i) * L + lane, mask=m)
                ncs.append(jnp.minimum(cq + nm, jnp.int32(QCAP)))
            return tuple(ncs)

        qcnts = lax.fori_loop(0, QV, coll,
                              tuple(jnp.int32(0) for _ in range(Q)))
        nvq = []
        for q in range(Q):
            ck_v[pl.ds(q * QSTR + qcnts[q], L)] = jnp.full(
                (L,), MIN_I32, jnp.int32)
            nvq.append(qcnts[q] // L + 1)

        NEG_INF = jnp.float32(-jnp.inf)

        def ext(j, carry):
            lk, lp = carry

            def make_scan(q):
                def scanmax(t, mc):
                    mv, pv = mc
                    kv = ck_v[pl.ds(q * QSTR + t * L, L)]
                    pos = q * QSTR + t * L + lane
                    elig = jnp.logical_or(
                        kv < lk, jnp.logical_and(kv == lk, pos > lp))
                    kv2 = jnp.where(elig, kv, jnp.int32(MIN_I32))
                    upd = kv2 > mv
                    mv = jnp.where(upd, kv2, mv)
                    pv = jnp.where(upd, pos, pv)
                    return (mv, pv)
                return scanmax

            mc = (jnp.full((L,), MIN_I32, jnp.int32),
                  jnp.full((L,), BIG_I32, jnp.int32))
            for q in range(Q):
                mc = lax.fori_loop(0, nvq[q], make_scan(q), mc)
            mv, pv = mc
            m_key = jnp.max(mv)
            p = jnp.min(jnp.where(mv == m_key, pv, jnp.int32(BIG_I32)))
            plsc.store_compressed(p96.at[pl.ds(j, L)], _splat(p), mask=lane0)
            return (m_key, p)

        lax.fori_loop(0, P, ext, (jnp.int32(0x7FFFFFFF), jnp.int32(-1)))

        def pfetch(t, c):
            pos = p96[pl.ds(t * L, L)]
            v96[pl.ds(t * L, L)] = plsc.load_gather(cv_v, [pos])
            i96[pl.ds(t * L, L)] = plsc.load_gather(ci_v, [pos])
            return c

        lax.fori_loop(0, P // L, pfetch, 0)

        hyp_r = hyp_v[pl.ds(row, L)][0]

        def gcomp(t, c):
            g = (v96[pl.ds(t * L, L)]
                 + jnp.float32(W_PART) * part_v[pl.ds(t * L, L)] + hyp_r)
            g96[pl.ds(t * L, L)] = g
            k2[pl.ds(t * L, L)] = _skey(g)
            return c

        lax.fori_loop(0, P // L, gcomp, 0)

        def ext2(j, carry):
            lk, lp = carry

            def scanmax2(t, mc):
                mv, pv = mc
                kv = k2[pl.ds(t * L, L)]
                pos = t * L + lane
                elig = jnp.logical_or(
                    kv < lk, jnp.logical_and(kv == lk, pos > lp))
                kv2 = jnp.where(elig, kv, jnp.int32(MIN_I32))
                upd = kv2 > mv
                mv = jnp.where(upd, kv2, mv)
                pv = jnp.where(upd, pos, pv)
                return (mv, pv)

            mv, pv = lax.fori_loop(
                0, P // L, scanmax2,
                (jnp.full((L,), MIN_I32, jnp.int32),
                 jnp.full((L,), BIG_I32, jnp.int32)))
            m_key = jnp.max(mv)
            p = jnp.min(jnp.where(mv == m_key, pv, jnp.int32(BIG_I32)))
            plsc.store_compressed(ol.at[pl.ds(j, L)], _splat(p), mask=lane0)
            return (m_key, p)

        lax.fori_loop(0, K, ext2, (jnp.int32(0x7FFFFFFF), jnp.int32(-1)))

        def ofetch(t, c):
            pos = ol[pl.ds(t * L, L)]
            ovx[pl.ds(t * L, L)] = plsc.load_gather(g96, [pos])
            oix[pl.ds(t * L, L)] = plsc.load_gather(i96, [pos])
            olx[pl.ds(t * L, L)] = pos
            return c

        lax.fori_loop(0, K // L, ofetch, 0)
        pltpu.sync_copy(ovx, ovals_hbm.at[row])
        pltpu.sync_copy(oix, oids_hbm.at[row])
        pltpu.sync_copy(olx, olids_hbm.at[row])
        return _

    lax.fori_loop(0, RPW, do_row, 0)


def kernel(full_scores, part_scores, hyp_scores):
    return _beam_step(full_scores, part_scores, hyp_scores)


# speedup vs baseline: 1.7486x; 1.7486x over previous
import functools

import jax
import jax.numpy as jnp
from jax import lax
from jax.experimental import pallas as pl
from jax.experimental.pallas import tpu as pltpu
from jax.experimental.pallas import tpu_sc as plsc

B = 128
V = 32768
P = 96
K = 64
W_PART = 0.3

NC = 2
NS = 16
NW = NC * NS
RPW = B // NW
L = 16

NBINS = 1024
CAP = 2048
Q = 4
QV = V // L // Q
QCAP = 1024
QSTR = QCAP + L
MIN_I32 = -(2**31)
BIG_I32 = 2**30


def _skey(v):
    b = lax.bitcast_convert_type(v, jnp.int32)
    return jnp.where(b >= 0, b, b ^ jnp.int32(0x7FFFFFFF))


_mesh = plsc.VectorSubcoreMesh(core_axis_name="c", subcore_axis_name="s")


@functools.partial(
    pl.kernel,
    out_type=(
        jax.ShapeDtypeStruct((B, K), jnp.float32),
        jax.ShapeDtypeStruct((B, K), jnp.int32),
        jax.ShapeDtypeStruct((B, K), jnp.int32),
    ),
    mesh=_mesh,
    compiler_params=pltpu.CompilerParams(needs_layout_passes=False),
    scratch_types=[
        pltpu.VMEM((2 * V,), jnp.float32),
        pltpu.VMEM((NBINS * L,), jnp.int32),
        pltpu.VMEM((Q * QSTR,), jnp.int32),
        pltpu.VMEM((Q * QSTR,), jnp.float32),
        pltpu.VMEM((Q * QSTR,), jnp.int32),
        pltpu.VMEM((P + L,), jnp.float32),
        pltpu.VMEM((P + L,), jnp.int32),
        pltpu.VMEM((P + L,), jnp.int32),
        pltpu.VMEM((P + L,), jnp.float32),
        pltpu.VMEM((P + L,), jnp.int32),
        pltpu.VMEM((P,), jnp.float32),
        pltpu.VMEM((B + L,), jnp.float32),
        pltpu.VMEM((K + L,), jnp.float32),
        pltpu.VMEM((K + L,), jnp.int32),
        pltpu.VMEM((K + L,), jnp.int32),
        pltpu.VMEM((K,), jnp.float32),
        pltpu.VMEM((K,), jnp.int32),
        pltpu.VMEM((K,), jnp.int32),
        pltpu.SemaphoreType.DMA,
    ],
)
def _beam_step(full_hbm, part_hbm, hyp_hbm,
               ovals_hbm, oids_hbm, olids_hbm,
               row_v, hist_v, ck_v, cv_v, ci_v,
               v96, i96, p96, g96, k2, part_v, hyp_v,
               ov, oi, ol, ovx, oix, olx, dsem):
    wid = lax.axis_index("s") * NC + lax.axis_index("c")
    lane = lax.iota(jnp.int32, L)
    lane0 = lane == 0
    pltpu.sync_copy(hyp_hbm, hyp_v.at[pl.ds(0, B)])

    def _splat(x):
        return jnp.full((L,), x)

    pltpu.async_copy(full_hbm.at[wid * RPW], row_v.at[pl.ds(0, V)], dsem)

    def do_row(r, _):
        row = wid * RPW + r
        base = (r % 2) * V
        pltpu.make_async_copy(
            full_hbm.at[row], row_v.at[pl.ds(base, V)], dsem).wait()

        @pl.when(r + 1 < RPW)
        def _prefetch():
            pltpu.async_copy(
                full_hbm.at[row + 1],
                row_v.at[pl.ds(((r + 1) % 2) * V, V)], dsem)

        pltpu.sync_copy(part_hbm.at[row], part_v)

        zeros = jnp.zeros((L,), jnp.int32)

        def zro(i, c):
            hist_v[pl.ds(i * L, L)] = zeros
            return c

        lax.fori_loop(0, NBINS, zro, 0, unroll=8)

        ones = jnp.ones((L,), jnp.int32)

        def hist_body(i, mx):
            kk = _skey(row_v[pl.ds(base + i * L, L)])
            bins = lax.shift_right_arithmetic(kk, 22) + 512
            plsc.addupdate_scatter(hist_v, [bins * L + lane], ones)
            return jnp.maximum(mx, kk)

        mxv = lax.fori_loop(0, V // L, hist_body,
                            jnp.full((L,), MIN_I32, jnp.int32), unroll=4)
        top_bin = lax.shift_right_arithmetic(jnp.max(mxv), 22) + 512

        def t_cond(c):
            b, acc = c
            return jnp.logical_and(acc < P, b > 0)

        def t_body(c):
            b, acc = c
            b = b - 1
            acc = acc + jnp.sum(hist_v[pl.ds(b * L, L)])
            return (b, acc)

        thr_bin, _ = lax.while_loop(t_cond, t_body,
                                    (top_bin + 1, jnp.int32(0)))

        thr_key = lax.shift_left(thr_bin - 512, 22)

        def coll(i, cs):
            ncs = []
            for q in range(Q):
                cq = cs[q]
                v = row_v[pl.ds(base + (q * QV + i) * L, L)]
                kk = _skey(v)
                m = kk >= thr_key
                nm = plsc.all_reduce_population_count(m)[0]
                plsc.store_compressed(ck_v.at[pl.ds(q * QSTR + cq, L)], kk,
                                      mask=m)
                plsc.store_compressed(cv_v.at[pl.ds(q * QSTR + cq, L)], v,
                                      mask=m)
                plsc.store_compressed(ci_v.at[pl.ds(q * QSTR + cq, L)],
                                      (q * QV + i) * L + lane, mask=m)
                ncs.append(jnp.minimum(cq + nm, jnp.int32(QCAP)))
            return tuple(ncs)

        qcnts = lax.fori_loop(0, QV, coll,
                              tuple(jnp.int32(0) for _ in range(Q)))
        nvq = []
        for q in range(Q):
            ck_v[pl.ds(q * QSTR + qcnts[q], L)] = jnp.full(
                (L,), MIN_I32, jnp.int32)
            nvq.append(qcnts[q] // L + 1)

        NEG_INF = jnp.float32(-jnp.inf)

        def ext(j, carry):
            lk, lp = carry

            def make_scan(q):
                def scanmax(t, mc):
                    mv, pv = mc
                    kv = ck_v[pl.ds(q * QSTR + t * L, L)]
                    pos = q * QSTR + t * L + lane
                    elig = jnp.logical_or(
                        kv < lk, jnp.logical_and(kv == lk, pos > lp))
                    kv2 = jnp.where(elig, kv, jnp.int32(MIN_I32))
                    upd = kv2 > mv
                    mv = jnp.where(upd, kv2, mv)
                    pv = jnp.where(upd, pos, pv)
                    return (mv, pv)
                return scanmax

            mc = (jnp.full((L,), MIN_I32, jnp.int32),
                  jnp.full((L,), BIG_I32, jnp.int32))
            for q in range(Q):
                mc = lax.fori_loop(0, nvq[q], make_scan(q), mc)
            mv, pv = mc
            m_key = jnp.max(mv)
            p = jnp.min(jnp.where(mv == m_key, pv, jnp.int32(BIG_I32)))
            plsc.store_compressed(p96.at[pl.ds(j, L)], _splat(p), mask=lane0)
            return (m_key, p)

        lax.fori_loop(0, P, ext, (jnp.int32(0x7FFFFFFF), jnp.int32(-1)), unroll=2)

        def pfetch(t, c):
            pos = p96[pl.ds(t * L, L)]
            v96[pl.ds(t * L, L)] = plsc.load_gather(cv_v, [pos])
            i96[pl.ds(t * L, L)] = plsc.load_gather(ci_v, [pos])
            return c

        lax.fori_loop(0, P // L, pfetch, 0, unroll=6)

        hyp_r = hyp_v[pl.ds(row, L)][0]

        def gcomp(t, c):
            g = (v96[pl.ds(t * L, L)]
                 + jnp.float32(W_PART) * part_v[pl.ds(t * L, L)] + hyp_r)
            g96[pl.ds(t * L, L)] = g
            k2[pl.ds(t * L, L)] = _skey(g)
            return c

        lax.fori_loop(0, P // L, gcomp, 0)

        def ext2(j, carry):
            lk, lp = carry

            def scanmax2(t, mc):
                mv, pv = mc
                kv = k2[pl.ds(t * L, L)]
                pos = t * L + lane
                elig = jnp.logical_or(
                    kv < lk, jnp.logical_and(kv == lk, pos > lp))
                kv2 = jnp.where(elig, kv, jnp.int32(MIN_I32))
                upd = kv2 > mv
                mv = jnp.where(upd, kv2, mv)
                pv = jnp.where(upd, pos, pv)
                return (mv, pv)

            mv, pv = lax.fori_loop(
                0, P // L, scanmax2,
                (jnp.full((L,), MIN_I32, jnp.int32),
                 jnp.full((L,), BIG_I32, jnp.int32)), unroll=6)
            m_key = jnp.max(mv)
            p = jnp.min(jnp.where(mv == m_key, pv, jnp.int32(BIG_I32)))
            plsc.store_compressed(ol.at[pl.ds(j, L)], _splat(p), mask=lane0)
            return (m_key, p)

        lax.fori_loop(0, K, ext2, (jnp.int32(0x7FFFFFFF), jnp.int32(-1)), unroll=2)

        def ofetch(t, c):
            pos = ol[pl.ds(t * L, L)]
            ovx[pl.ds(t * L, L)] = plsc.load_gather(g96, [pos])
            oix[pl.ds(t * L, L)] = plsc.load_gather(i96, [pos])
            olx[pl.ds(t * L, L)] = pos
            return c

        lax.fori_loop(0, K // L, ofetch, 0, unroll=4)
        pltpu.sync_copy(ovx, ovals_hbm.at[row])
        pltpu.sync_copy(oix, oids_hbm.at[row])
        pltpu.sync_copy(olx, olids_hbm.at[row])
        return _

    lax.fori_loop(0, RPW, do_row, 0)


def kernel(full_scores, part_scores, hyp_scores):
    return _beam_step(full_scores, part_scores, hyp_scores)
